# baseline (speedup 1.0000x reference)
import functools

import jax
import jax.numpy as jnp
from jax import lax
from jax.experimental import pallas as pl
from jax.experimental.pallas import tpu as pltpu
from jax.experimental.pallas import tpu_sc as plsc

_VOCAB = 100000
_D = 64
_B = 4096
_T = 50
_NW = 32
_BPW = _B // _NW
_GB = 16
_G = _BPW // _GB


def _emb_body(table_hbm, idx_hbm, out_hbm, idx_v, rows_v, gsem, ssem):
    wid = lax.axis_index("s") * 2 + lax.axis_index("c")
    base = wid * _BPW
    pltpu.sync_copy(idx_hbm.at[wid], idx_v)

    def fire_gathers(g, phase):
        for j in range(_GB):
            pltpu.async_copy(
                table_hbm.at[idx_v.at[g * _GB + j]],
                rows_v.at[phase, j],
                gsem,
            )

    def wait_gathers():
        for _ in range(_GB):
            pltpu.make_async_copy(
                table_hbm.at[idx_v.at[0]],
                rows_v.at[0, 0],
                gsem,
            ).wait()

    def wait_store():
        pltpu.make_async_copy(
            rows_v.at[0],
            out_hbm.at[pl.ds(0, _GB)],
            ssem,
        ).wait()

    fire_gathers(0, 0)

    def body(g, carry):
        phase = lax.rem(g, 2)
        wait_gathers()

        @pl.when(g > 0)
        def _():
            wait_store()

        pltpu.async_copy(
            rows_v.at[phase],
            out_hbm.at[pl.ds(base + g * _GB, _GB)],
            ssem,
        )

        @pl.when(g < _G - 1)
        def _():
            fire_gathers(g + 1, 1 - phase)

        return carry

    lax.fori_loop(0, _G, body, 0)
    wait_store()


_emb_call = functools.partial(
    pl.kernel,
    mesh=plsc.VectorSubcoreMesh(core_axis_name="c", subcore_axis_name="s"),
    out_type=jax.ShapeDtypeStruct((_B, _T, _D), jnp.float32),
    scratch_types=[
        pltpu.VMEM((_BPW, _T), jnp.int32),
        pltpu.VMEM((2, _GB, _T, _D), jnp.float32),
        pltpu.SemaphoreType.DMA,
        pltpu.SemaphoreType.DMA,
    ],
    compiler_params=pltpu.CompilerParams(use_tc_tiling_on_sc=False),
)(_emb_body)


@jax.jit
def kernel(input, emb_weight):
    idx = input.reshape(_NW, _BPW, _T).astype(jnp.int32)
    return _emb_call(emb_weight, idx)
